# baseline (speedup 1.0000x reference)
import jax
import jax.numpy as jnp
from jax.experimental import pallas as pl
from jax.experimental.pallas import tpu as pltpu
from jax.experimental.pallas import tpu_sc as plsc


def kernel(x, pos_table):
    B, L, D = x.shape
    N = B * L
    x2 = x.reshape(N, D)
    pos = pos_table[:L]
    RB = 16
    mesh = plsc.VectorSubcoreMesh(core_axis_name="core", subcore_axis_name="subcore")

    @pl.kernel(out_type=jax.ShapeDtypeStruct((N, D), x.dtype), mesh=mesh)
    def sc_add(x_hbm, p_hbm, o_hbm):
        def body(x_vmem, p_vmem, o_vmem):
            @pl.loop(0, RB)
            def _(r):
                @pl.loop(0, D, step=16)
                def _(c):
                    slc = (pl.ds(r, 1), pl.ds(c, 16))
                    o_vmem.at[*slc][...] = (
                        x_vmem.at[*slc][...] + p_vmem.at[*slc][...]
                    )

        pltpu.emit_pipeline(
            body,
            grid=(N // RB,),
            in_specs=[
                pl.BlockSpec((RB, D), index_map=lambda i: (i, 0)),
                pl.BlockSpec((RB, D), index_map=lambda i: (i % (L // RB), 0)),
            ],
            out_specs=[pl.BlockSpec((RB, D), index_map=lambda i: (i, 0))],
            core_axis_name=("core", "subcore"),
            dimension_semantics=(pltpu.PARALLEL,),
        )(x_hbm, p_hbm, o_hbm)

    return sc_add(x2, pos).reshape(B, L, D)


# speedup vs baseline: 4.3535x; 4.3535x over previous
import jax
import jax.numpy as jnp
from jax.experimental import pallas as pl


def _add_kernel(x_ref, p_ref, o_ref):
    o_ref[...] = x_ref[...] + p_ref[...]


def kernel(x, pos_table):
    B, L, D = x.shape
    TL = 2048
    nj = L // TL
    x2 = x.reshape(B * L, D)
    out = pl.pallas_call(
        _add_kernel,
        grid=(B * nj,),
        in_specs=[
            pl.BlockSpec((TL, D), lambda i: ((i % B) * nj + i // B, 0)),
            pl.BlockSpec((TL, D), lambda i: (i // B, 0)),
        ],
        out_specs=pl.BlockSpec((TL, D), lambda i: ((i % B) * nj + i // B, 0)),
        out_shape=jax.ShapeDtypeStruct((B * L, D), x.dtype),
    )(x2, pos_table[:L])
    return out.reshape(B, L, D)


# speedup vs baseline: 4.3666x; 1.0030x over previous
import jax
import jax.numpy as jnp
from jax.experimental import pallas as pl
from jax.experimental.pallas import tpu as pltpu

CT = 512
PJ = 2048
K = 4


def _make_kernel(B, L, D):
    NJ = L // PJ
    NC = PJ // CT
    S = NJ * B * NC

    def body(x_hbm, p_hbm, o_hbm, pbuf, xbuf, obuf, psem, isem, osem):
        def row_start(s):
            j = s // (B * NC)
            b = (s // NC) % B
            c = s % NC
            return b * L + j * PJ + c * CT, j, c

        def start_in(s):
            r, _, _ = row_start(s)
            pltpu.make_async_copy(
                x_hbm.at[pl.ds(r, CT), :], xbuf.at[s % K], isem.at[s % K]
            ).start()

        def wait_in(s):
            r, _, _ = row_start(s)
            pltpu.make_async_copy(
                x_hbm.at[pl.ds(r, CT), :], xbuf.at[s % K], isem.at[s % K]
            ).wait()

        def start_pos(j):
            pltpu.make_async_copy(
                p_hbm.at[pl.ds(j * PJ, PJ), :], pbuf.at[j % 2], psem.at[j % 2]
            ).start()

        def wait_pos(j):
            pltpu.make_async_copy(
                p_hbm.at[pl.ds(j * PJ, PJ), :], pbuf.at[j % 2], psem.at[j % 2]
            ).wait()

        def start_out(s):
            r, _, _ = row_start(s)
            pltpu.make_async_copy(
                obuf.at[s % K], o_hbm.at[pl.ds(r, CT), :], osem.at[s % K]
            ).start()

        def wait_out(s):
            r, _, _ = row_start(s)
            pltpu.make_async_copy(
                obuf.at[s % K], o_hbm.at[pl.ds(r, CT), :], osem.at[s % K]
            ).wait()

        start_pos(0)
        for s0 in range(K):
            start_in(s0)

        def step(s, _):
            m = s % K
            _, j, c = row_start(s)

            @pl.when(jnp.logical_and(s % (B * NC) == 0, j + 1 < NJ))
            def _():
                start_pos(j + 1)

            @pl.when(s % (B * NC) == 0)
            def _():
                wait_pos(j)

            wait_in(s)

            @pl.when(s >= K)
            def _():
                wait_out(s - K)

            obuf[m] = xbuf[m] + pbuf[j % 2, pl.ds(c * CT, CT), :]
            start_out(s)

            @pl.when(s + K < S)
            def _():
                start_in(s + K)

            return None

        jax.lax.fori_loop(0, S, step, None)

        for st in range(S - K, S):
            wait_out(st)

    return body, S


def kernel(x, pos_table):
    B, L, D = x.shape
    body, _ = _make_kernel(B, L, D)
    x2 = x.reshape(B * L, D)
    out = pl.pallas_call(
        body,
        in_specs=[
            pl.BlockSpec(memory_space=pltpu.MemorySpace.HBM),
            pl.BlockSpec(memory_space=pltpu.MemorySpace.HBM),
        ],
        out_specs=pl.BlockSpec(memory_space=pltpu.MemorySpace.HBM),
        out_shape=jax.ShapeDtypeStruct((B * L, D), x.dtype),
        scratch_shapes=[
            pltpu.VMEM((2, PJ, D), x.dtype),
            pltpu.VMEM((K, CT, D), x.dtype),
            pltpu.VMEM((K, CT, D), x.dtype),
            pltpu.SemaphoreType.DMA((2,)),
            pltpu.SemaphoreType.DMA((K,)),
            pltpu.SemaphoreType.DMA((K,)),
        ],
    )(x2, pos_table[:L])
    return out.reshape(B, L, D)
